# Initial kernel scaffold; baseline (speedup 1.0000x reference)
import jax
import jax.numpy as jnp
from jax.experimental import pallas as pl


def kernel(x, edge_index, batch, params):
    raise NotImplementedError("write your pallas kernel here")


# speedup vs baseline: 1.0262x; 1.0262x over previous
import functools
import math

import jax
import jax.numpy as jnp
from jax.experimental import pallas as pl
from jax.experimental.pallas import tpu as pltpu

N_NODES = 10000
N_EDGES = 320000
HID = 128
NUM_LAYERS = 3
_AVG_LOG = math.log(33.0)

_BLK = 1000
_GRID = N_NODES // _BLK


def _dense_body(final, s_ref, ssq_ref, mx_ref, mn_ref, deg_ref, h_ref,
                wa_ref, wb_ref, wc_ref, cb_ref, bn_ref, fcw_ref, fcb_ref,
                wih_ref, whh_ref, bih_ref, bhh_ref, lw_ref, lb_ref,
                hout_ref, out_ref):
    deg = deg_ref[...]
    degc = jnp.maximum(deg, 1.0)
    inv = 1.0 / degc
    mean = s_ref[...] * inv
    var = jnp.maximum(ssq_ref[...] * inv - mean * mean, 0.0)
    std = jnp.sqrt(var + 1e-5)
    has = deg > 0.0
    mx = jnp.where(has, mx_ref[...], 0.0)
    mn = jnp.where(has, mn_ref[...], 0.0)
    agg = jnp.concatenate([mean, mn, mx, std], axis=1)
    logd = jnp.log(degc + 1.0)
    s1 = logd * (1.0 / _AVG_LOG)
    s2 = _AVG_LOG / logd
    x = (jnp.dot(agg, wa_ref[...], preferred_element_type=jnp.float32)
         + jnp.dot(agg * s1, wb_ref[...], preferred_element_type=jnp.float32)
         + jnp.dot(agg * s2, wc_ref[...], preferred_element_type=jnp.float32)
         + cb_ref[...])
    x = x * bn_ref[0:1, :] + bn_ref[1:2, :]
    x = jnp.dot(x, fcw_ref[...], preferred_element_type=jnp.float32) + fcb_ref[...]
    x = jnp.maximum(x, 0.0)
    h = h_ref[...]
    gi = jnp.dot(x, wih_ref[...], preferred_element_type=jnp.float32) + bih_ref[...]
    gh = jnp.dot(h, whh_ref[...], preferred_element_type=jnp.float32) + bhh_ref[...]
    i_r, i_z, i_n = gi[:, :HID], gi[:, HID:2 * HID], gi[:, 2 * HID:]
    h_r, h_z, h_n = gh[:, :HID], gh[:, HID:2 * HID], gh[:, 2 * HID:]
    r = jax.nn.sigmoid(i_r + h_r)
    z = jax.nn.sigmoid(i_z + h_z)
    ng = jnp.tanh(i_n + r * h_n)
    hnew = (1.0 - z) * ng + z * h
    hout_ref[...] = hnew
    if final:
        out_ref[...] = (jnp.dot(hnew, lw_ref[...],
                                preferred_element_type=jnp.float32) + lb_ref[...])


def _node_spec(cols):
    return pl.BlockSpec((_BLK, cols), lambda i: (i, 0))


def _rep_spec(shape):
    nd = len(shape)
    return pl.BlockSpec(shape, lambda i: (0,) * nd)


@functools.partial(jax.jit, static_argnames=("final",))
def _dense_layer(s, ssq, mx, mn, deg, h, wa, wb, wc, cb, bn, fcw, fcb,
                 wih, whh, bih, bhh, lw, lb, final=False):
    out_shapes = [jax.ShapeDtypeStruct((N_NODES, HID), jnp.float32),
                  jax.ShapeDtypeStruct((N_NODES, HID), jnp.float32)]
    in_specs = [_node_spec(HID)] * 4 + [_node_spec(1), _node_spec(HID)]
    in_specs += [_rep_spec(w.shape) for w in
                 (wa, wb, wc, cb, bn, fcw, fcb, wih, whh, bih, bhh, lw, lb)]
    outs = pl.pallas_call(
        functools.partial(_dense_body, final),
        grid=(_GRID,),
        in_specs=in_specs,
        out_specs=[_node_spec(HID), _node_spec(HID)],
        out_shape=out_shapes,
    )(s, ssq, mx, mn, deg, h, wa, wb, wc, cb, bn, fcw, fcb,
      wih, whh, bih, bhh, lw, lb)
    return outs


def kernel(x, edge_index, batch, params):
    src = edge_index[0]
    dst = edge_index[1]
    deg = jax.ops.segment_sum(jnp.ones((N_EDGES,), jnp.float32), dst,
                              num_segments=N_NODES)[:, None]
    h = jnp.zeros((N_NODES, HID), jnp.float32)
    bih = params['gru_b_ih'][None, :]
    bhh = params['gru_b_hh'][None, :]
    wih = params['gru_w_ih'].T
    whh = params['gru_w_hh'].T
    lw = params['last_w'].T
    lb = params['last_b'][None, :]
    out = None
    for i in range(NUM_LAYERS):
        msg = x[src]
        s = jax.ops.segment_sum(msg, dst, num_segments=N_NODES)
        ssq = jax.ops.segment_sum(msg * msg, dst, num_segments=N_NODES)
        mx = jax.ops.segment_max(msg, dst, num_segments=N_NODES)
        mn = jax.ops.segment_min(msg, dst, num_segments=N_NODES)
        w = params['conv%d_w' % i]
        wa = w[:, :512].T
        wb = w[:, 512:1024].T
        wc = w[:, 1024:].T
        cb = params['conv%d_b' % i][None, :]
        g = params['bn%d_g' % i] / math.sqrt(1.0 + 1e-5)
        bn = jnp.stack([g, params['bn%d_b' % i]], axis=0)
        fcw = params['fc%d_w' % i].T
        fcb = params['fc%d_b' % i][None, :]
        final = i == NUM_LAYERS - 1
        h, out = _dense_layer(s, ssq, mx, mn, deg, h, wa, wb, wc, cb, bn,
                              fcw, fcb, wih, whh, bih, bhh, lw, lb,
                              final=final)
        x = h
    return out


# speedup vs baseline: 3.9765x; 3.8751x over previous
import functools
import math

import jax
import jax.numpy as jnp
from jax import lax
from jax.experimental import pallas as pl
from jax.experimental.pallas import tpu as pltpu
from jax.experimental.pallas import tpu_sc as plsc

N_NODES = 10000
N_EDGES = 320000
HID = 128
NUM_LAYERS = 3
_AVG_LOG = math.log(33.0)

_NT = 32
_NPT = 320
_NPAD = _NT * _NPT
_SPR = 328
_CAP = 11264
_PAD_ENC = _NPT
_CH = 3200
_G = 64

_sc_mesh = plsc.VectorSubcoreMesh(core_axis_name="c", subcore_axis_name="s")
_sc_params = pltpu.CompilerParams(needs_layout_passes=False)


def _wid():
    return lax.axis_index("c") * 16 + lax.axis_index("s")


@functools.partial(
    pl.kernel,
    out_type=[
        jax.ShapeDtypeStruct((_NT, 1, _CAP), jnp.int32),
        jax.ShapeDtypeStruct((_NT, 1, 16), jnp.int32),
        jax.ShapeDtypeStruct((_NT, 1, 336), jnp.float32),
    ],
    mesh=_sc_mesh,
    scratch_types=[
        pltpu.VMEM((_CAP + 16,), jnp.int32),
        pltpu.VMEM((_CH,), jnp.int32),
        pltpu.VMEM((_CH,), jnp.int32),
        pltpu.VMEM((336,), jnp.float32),
        pltpu.VMEM((16,), jnp.int32),
    ],
    compiler_params=_sc_params,
)
def _route(src_hbm, dst_hbm, owned_hbm, cnt_hbm, deg_hbm,
           owned_v, src_v, dst_v, deg_v, misc_v):
    wid = _wid()
    lo = wid * _NPT
    pad16 = jnp.full((16,), _PAD_ENC, jnp.int32)

    def initb(i, carry):
        owned_v[pl.ds(i * 16, 16)] = pad16
        return carry

    lax.fori_loop(0, (_CAP + 16) // 16, initb, 0)
    zero16 = jnp.zeros((16,), jnp.float32)
    for i in range(21):
        deg_v[pl.ds(i * 16, 16)] = zero16

    def chunk_body(g, cnt):
        pltpu.sync_copy(src_hbm.at[pl.ds(g * _CH, _CH)], src_v)
        pltpu.sync_copy(dst_hbm.at[pl.ds(g * _CH, _CH)], dst_v)

        def scan16(i, cnt):
            dv = dst_v[pl.ds(i * 16, 16)]
            sv = src_v[pl.ds(i * 16, 16)]
            dl = dv - lo
            m = jnp.logical_and(dl >= 0, dl < _NPT)
            enc = sv * 512 + dl
            plsc.store_compressed(owned_v.at[pl.ds(cnt, 16)], enc, mask=m)
            return cnt + plsc.all_reduce_population_count(m)[0]

        return lax.fori_loop(0, _CH // 16, scan16, cnt)

    cnt = lax.fori_loop(0, N_EDGES // _CH, chunk_body, jnp.int32(0))

    ones16 = jnp.ones((16,), jnp.float32)

    def degb(i, carry):
        enc = owned_v[pl.ds(i * 16, 16)]
        dl = jnp.bitwise_and(enc, 511)
        plsc.addupdate_scatter(deg_v, [dl], ones16)
        return carry

    lax.fori_loop(0, (cnt + 15) // 16, degb, 0)

    misc_v[pl.ds(0, 16)] = jnp.full((16,), cnt, jnp.int32)
    pltpu.sync_copy(owned_v.at[pl.ds(0, _CAP)], owned_hbm.at[wid, 0])
    pltpu.sync_copy(misc_v.at[pl.ds(0, 16)], cnt_hbm.at[wid, 0])
    pltpu.sync_copy(deg_v, deg_hbm.at[wid, 0])


def _make_reduce(p):
    @functools.partial(
        pl.kernel,
        out_type=[
            jax.ShapeDtypeStruct((_NPAD, 128), jnp.float32),
            jax.ShapeDtypeStruct((_NPAD, 128), jnp.float32),
        ],
        mesh=_sc_mesh,
        scratch_types=[
            pltpu.VMEM((_SPR, 128), jnp.float32),
            pltpu.VMEM((_SPR, 128), jnp.float32),
            pltpu.VMEM((2, _G, 128), jnp.float32),
            pltpu.VMEM((2, _G), jnp.int32),
            pltpu.VMEM((2, _G), jnp.int32),
            pltpu.VMEM((16,), jnp.int32),
            pltpu.SemaphoreType.DMA,
            pltpu.SemaphoreType.DMA,
        ],
        compiler_params=_sc_params,
    )
    def _reduce_p(xcat_hbm, owned_hbm, cnt_hbm, s2_hbm, mm_hbm,
                  sum_v, mm_v, gbuf, encb, srcb, cntv, gsem0, gsem1):
        wid = _wid()
        gsem = (gsem0, gsem1)
        obase = wid * _NPT

        pltpu.sync_copy(cnt_hbm.at[wid, 0], cntv)
        cnt = cntv[pl.ds(0, 16)][0]
        nch = (cnt + _G - 1) // _G

        ninf = jnp.full((16,), -jnp.inf, jnp.float32)
        pinf = jnp.full((16,), jnp.inf, jnp.float32)
        zero16 = jnp.zeros((16,), jnp.float32)

        def gather_cp(slot):
            return pltpu.make_async_copy(xcat_hbm.at[srcb.at[slot]],
                                         gbuf.at[slot], gsem[slot])

        def initb(i, carry):
            for k in range(4):
                sum_v[i, pl.ds(k * 16, 16)] = zero16
                sum_v[i, pl.ds(64 + k * 16, 16)] = zero16
                mm_v[i, pl.ds(k * 16, 16)] = ninf
                mm_v[i, pl.ds(64 + k * 16, 16)] = pinf
            return carry

        lax.fori_loop(0, _SPR, initb, 0)

        def prep(g, slot):
            pltpu.sync_copy(owned_hbm.at[wid, 0, pl.ds(g * _G, _G)],
                            encb.at[slot])
            for q in range(_G // 16):
                enc = encb[slot, pl.ds(q * 16, 16)]
                srcb[slot, pl.ds(q * 16, 16)] = (
                    jnp.right_shift(enc, 9) * 2 + p)

        @pl.when(nch >= 1)
        def _():
            prep(0, 0)
            gather_cp(0).start()

        def outer(gg, carry):
            for b in range(2):
                g = gg * 2 + b
                slot, nxt = b, 1 - b

                @pl.when(g < nch)
                def _():
                    @pl.when(g + 1 < nch)
                    def _():
                        prep(g + 1, nxt)
                        gather_cp(nxt).start()

                    gather_cp(slot).wait()

                    def rmw(q, carry):
                        enc = encb[slot, pl.ds(q * 16, 16)]
                        for j in range(16):
                            dl = jnp.bitwise_and(enc[j], 511)
                            row = q * 16 + j
                            for k in range(4):
                                cs = pl.ds(k * 16, 16)
                                c2 = pl.ds(64 + k * 16, 16)
                                m = gbuf[slot, row, cs]
                                m2 = gbuf[slot, row, c2]
                                sum_v[dl, cs] = sum_v[dl, cs] + m
                                sum_v[dl, c2] = sum_v[dl, c2] + m2
                                mm_v[dl, cs] = jnp.maximum(mm_v[dl, cs], m)
                                mm_v[dl, c2] = jnp.minimum(mm_v[dl, c2], m)
                        return carry

                    lax.fori_loop(0, _G // 16, rmw, 0)

            return carry

        lax.fori_loop(0, (nch + 1) // 2, outer, 0)

        pltpu.sync_copy(sum_v.at[pl.ds(0, _NPT)],
                        s2_hbm.at[pl.ds(obase, _NPT)])
        pltpu.sync_copy(mm_v.at[pl.ds(0, _NPT)],
                        mm_hbm.at[pl.ds(obase, _NPT)])

    return _reduce_p


_reduce_half = (_make_reduce(0), _make_reduce(1))


_BLK = 1000
_GRID = N_NODES // _BLK


def _prep_body(x_ref, o_ref):
    x = x_ref[...]
    for p in range(2):
        xh = x[:, 64 * p:64 * p + 64]
        o_ref[:, p, 0:64] = xh
        o_ref[:, p, 64:128] = xh * xh


@jax.jit
def _prep(x):
    return pl.pallas_call(
        _prep_body,
        grid=(_GRID,),
        in_specs=[pl.BlockSpec((_BLK, HID), lambda i: (i, 0))],
        out_specs=pl.BlockSpec((_BLK, 2, 128), lambda i: (i, 0, 0)),
        out_shape=jax.ShapeDtypeStruct((N_NODES, 2, 128), jnp.float32),
    )(x)


def _dense_body(final, s_ref, ssq_ref, mx_ref, mn_ref, deg_ref, h_ref,
                wa_ref, wb_ref, wc_ref, cb_ref, bn_ref, fcw_ref, fcb_ref,
                wih_ref, whh_ref, bih_ref, bhh_ref, lw_ref, lb_ref,
                hcat_ref, out_ref):
    deg = deg_ref[...]
    degc = jnp.maximum(deg, 1.0)
    inv = 1.0 / degc
    mean = s_ref[...] * inv
    var = jnp.maximum(ssq_ref[...] * inv - mean * mean, 0.0)
    std = jnp.sqrt(var + 1e-5)
    has = deg > 0.0
    mx = jnp.where(has, mx_ref[...], 0.0)
    mn = jnp.where(has, mn_ref[...], 0.0)
    agg = jnp.concatenate([mean, mn, mx, std], axis=1)
    logd = jnp.log(degc + 1.0)
    s1 = logd * (1.0 / _AVG_LOG)
    s2 = _AVG_LOG / logd
    x = (jnp.dot(agg, wa_ref[...], preferred_element_type=jnp.float32)
         + jnp.dot(agg * s1, wb_ref[...], preferred_element_type=jnp.float32)
         + jnp.dot(agg * s2, wc_ref[...], preferred_element_type=jnp.float32)
         + cb_ref[...])
    x = x * bn_ref[0:1, :] + bn_ref[1:2, :]
    x = jnp.dot(x, fcw_ref[...], preferred_element_type=jnp.float32) + fcb_ref[...]
    x = jnp.maximum(x, 0.0)
    h = h_ref[...]
    gi = jnp.dot(x, wih_ref[...], preferred_element_type=jnp.float32) + bih_ref[...]
    gh = jnp.dot(h, whh_ref[...], preferred_element_type=jnp.float32) + bhh_ref[...]
    i_r, i_z, i_n = gi[:, :HID], gi[:, HID:2 * HID], gi[:, 2 * HID:]
    h_r, h_z, h_n = gh[:, :HID], gh[:, HID:2 * HID], gh[:, 2 * HID:]
    r = jax.nn.sigmoid(i_r + h_r)
    z = jax.nn.sigmoid(i_z + h_z)
    ng = jnp.tanh(i_n + r * h_n)
    hnew = (1.0 - z) * ng + z * h
    for p in range(2):
        hh = hnew[:, 64 * p:64 * p + 64]
        hcat_ref[:, p, 0:64] = hh
        hcat_ref[:, p, 64:128] = hh * hh
    if final:
        out_ref[...] = (jnp.dot(hnew, lw_ref[...],
                                preferred_element_type=jnp.float32) + lb_ref[...])


def _node_spec(cols):
    return pl.BlockSpec((_BLK, cols), lambda i: (i, 0))


def _rep_spec(shape):
    nd = len(shape)
    return pl.BlockSpec(shape, lambda i: (0,) * nd)


@functools.partial(jax.jit, static_argnames=("final",))
def _dense_layer(s, ssq, mx, mn, deg, h, wa, wb, wc, cb, bn, fcw, fcb,
                 wih, whh, bih, bhh, lw, lb, final=False):
    out_shapes = [jax.ShapeDtypeStruct((N_NODES, 2, 128), jnp.float32),
                  jax.ShapeDtypeStruct((N_NODES, HID), jnp.float32)]
    in_specs = [_node_spec(HID)] * 4 + [_node_spec(1), _node_spec(HID)]
    in_specs += [_rep_spec(w.shape) for w in
                 (wa, wb, wc, cb, bn, fcw, fcb, wih, whh, bih, bhh, lw, lb)]
    return pl.pallas_call(
        functools.partial(_dense_body, final),
        grid=(_GRID,),
        in_specs=in_specs,
        out_specs=[pl.BlockSpec((_BLK, 2, 128), lambda i: (i, 0, 0)),
                   _node_spec(HID)],
        out_shape=out_shapes,
    )(s, ssq, mx, mn, deg, h, wa, wb, wc, cb, bn, fcw, fcb,
      wih, whh, bih, bhh, lw, lb)


def kernel(x, edge_index, batch, params):
    src = edge_index[0]
    dst = edge_index[1]
    owned, cnt2, degw = _route(src, dst)
    deg = degw[:, 0, :_NPT].reshape(-1)[:N_NODES, None]

    xcat = jnp.pad(_prep(x).reshape(2 * N_NODES, 128),
                   ((0, 2 * _NPAD - 2 * N_NODES), (0, 0)))
    h = jnp.zeros((N_NODES, HID), jnp.float32)
    bih = params['gru_b_ih'][None, :]
    bhh = params['gru_b_hh'][None, :]
    wih = params['gru_w_ih'].T
    whh = params['gru_w_hh'].T
    lw = params['last_w'].T
    lb = params['last_b'][None, :]
    out = None
    for i in range(NUM_LAYERS):
        halves = [_reduce_half[p](xcat, owned, cnt2) for p in range(2)]
        sfull = jnp.concatenate([halves[p][0][:N_NODES, :64]
                                 for p in range(2)], 1)
        ssq = jnp.concatenate([halves[p][0][:N_NODES, 64:]
                               for p in range(2)], 1)
        mx = jnp.concatenate([halves[p][1][:N_NODES, :64]
                              for p in range(2)], 1)
        mn = jnp.concatenate([halves[p][1][:N_NODES, 64:]
                              for p in range(2)], 1)
        w = params['conv%d_w' % i]
        wa = w[:, :512].T
        wb = w[:, 512:1024].T
        wc = w[:, 1024:].T
        cb = params['conv%d_b' % i][None, :]
        g = params['bn%d_g' % i] / math.sqrt(1.0 + 1e-5)
        bn = jnp.stack([g, params['bn%d_b' % i]], axis=0)
        fcw = params['fc%d_w' % i].T
        fcb = params['fc%d_b' % i][None, :]
        final = i == NUM_LAYERS - 1
        hcat, out = _dense_layer(
            sfull, ssq, mx, mn, deg, h, wa, wb, wc, cb, bn, fcw, fcb,
            wih, whh, bih, bhh, lw, lb, final=final)
        h = jnp.concatenate([hcat[:, p, :64] for p in range(2)], 1)
        if not final:
            xcat = jnp.pad(hcat.reshape(2 * N_NODES, 128),
                           ((0, 2 * _NPAD - 2 * N_NODES), (0, 0)))
    return out


# speedup vs baseline: 4.5047x; 1.1328x over previous
import functools
import math

import jax
import jax.numpy as jnp
from jax import lax
from jax.experimental import pallas as pl
from jax.experimental.pallas import tpu as pltpu
from jax.experimental.pallas import tpu_sc as plsc

N_NODES = 10000
N_EDGES = 320000
HID = 128
NUM_LAYERS = 3
_AVG_LOG = math.log(33.0)

_NT = 32
_NPT = 320
_NPAD = _NT * _NPT
_SPR = 328
_CAP = 11264
_PAD_ENC = _NPT
_CH = 3200
_G = 64

_sc_mesh = plsc.VectorSubcoreMesh(core_axis_name="c", subcore_axis_name="s")
_sc_params = pltpu.CompilerParams(needs_layout_passes=False)


def _wid():
    return lax.axis_index("c") * 16 + lax.axis_index("s")


@functools.partial(
    pl.kernel,
    out_type=[
        jax.ShapeDtypeStruct((_NT, 1, _CAP), jnp.int32),
        jax.ShapeDtypeStruct((_NT, 1, 16), jnp.int32),
        jax.ShapeDtypeStruct((_NT, 1, 336), jnp.float32),
    ],
    mesh=_sc_mesh,
    scratch_types=[
        pltpu.VMEM((_CAP + 16,), jnp.int32),
        pltpu.VMEM((_CAP + 16,), jnp.int32),
        pltpu.VMEM((_CH,), jnp.int32),
        pltpu.VMEM((_CH,), jnp.int32),
        pltpu.VMEM((336,), jnp.float32),
        pltpu.VMEM((336,), jnp.int32),
        pltpu.VMEM((16,), jnp.int32),
        pltpu.VMEM((16,), jnp.int32),
    ],
    compiler_params=_sc_params,
)
def _route(src_hbm, dst_hbm, owned_hbm, cnt_hbm, deg_hbm,
           owned_v, sorted_v, src_v, dst_v, deg_v, off_v, dlbuf, misc_v):
    wid = _wid()
    lo = wid * _NPT
    pad16 = jnp.full((16,), _PAD_ENC, jnp.int32)

    def initb(i, carry):
        owned_v[pl.ds(i * 16, 16)] = pad16
        sorted_v[pl.ds(i * 16, 16)] = pad16
        return carry

    lax.fori_loop(0, (_CAP + 16) // 16, initb, 0)
    zero16 = jnp.zeros((16,), jnp.float32)
    for i in range(21):
        deg_v[pl.ds(i * 16, 16)] = zero16

    def chunk_body(g, cnt):
        pltpu.sync_copy(src_hbm.at[pl.ds(g * _CH, _CH)], src_v)
        pltpu.sync_copy(dst_hbm.at[pl.ds(g * _CH, _CH)], dst_v)

        def scan16(i, cnt):
            dv = dst_v[pl.ds(i * 16, 16)]
            sv = src_v[pl.ds(i * 16, 16)]
            dl = dv - lo
            m = jnp.logical_and(dl >= 0, dl < _NPT)
            enc = sv * 512 + dl
            plsc.store_compressed(owned_v.at[pl.ds(cnt, 16)], enc, mask=m)
            return cnt + plsc.all_reduce_population_count(m)[0]

        return lax.fori_loop(0, _CH // 16, scan16, cnt)

    cnt = lax.fori_loop(0, N_EDGES // _CH, chunk_body, jnp.int32(0))

    ones16 = jnp.ones((16,), jnp.float32)

    def degb(i, carry):
        enc = owned_v[pl.ds(i * 16, 16)]
        dl = jnp.bitwise_and(enc, 511)
        plsc.addupdate_scatter(deg_v, [dl], ones16)
        return carry

    lax.fori_loop(0, (cnt + 15) // 16, degb, 0)

    iota = lax.iota(jnp.int32, 16)
    carry = jnp.int32(0)
    for i in range(21):
        v = deg_v[pl.ds(i * 16, 16)].astype(jnp.int32)
        cum = plsc.cumsum(v)
        off_v[pl.ds(i * 16, 16)] = cum - v + carry
        carry = carry + cum[15]

    ones16i = jnp.ones((16,), jnp.int32)

    def placeb(i, carry):
        enc = owned_v[pl.ds(i * 16, 16)]
        dl = jnp.bitwise_and(enc, 511)
        dlbuf[pl.ds(0, 16)] = dl
        rank = jnp.zeros((16,), jnp.int32)
        for sft in range(1, 16):
            msk = iota >= sft
            sh = plsc.load_gather(dlbuf, [iota - sft], mask=msk)
            eq = jnp.logical_and(sh == dl, msk)
            rank = rank + eq.astype(jnp.int32)
        pos = plsc.load_gather(off_v, [dl]) + rank
        plsc.store_scatter(sorted_v, [pos], enc)
        plsc.addupdate_scatter(off_v, [dl], ones16i)
        return carry

    lax.fori_loop(0, (cnt + 15) // 16, placeb, 0)

    misc_v[pl.ds(0, 16)] = jnp.full((16,), cnt, jnp.int32)
    pltpu.sync_copy(sorted_v.at[pl.ds(0, _CAP)], owned_hbm.at[wid, 0])
    pltpu.sync_copy(misc_v.at[pl.ds(0, 16)], cnt_hbm.at[wid, 0])
    pltpu.sync_copy(deg_v, deg_hbm.at[wid, 0])


def _make_reduce(p):
    @functools.partial(
        pl.kernel,
        out_type=[
            jax.ShapeDtypeStruct((_NPAD, 128), jnp.float32),
            jax.ShapeDtypeStruct((_NPAD, 128), jnp.float32),
        ],
        mesh=_sc_mesh,
        scratch_types=[
            pltpu.VMEM((_SPR, 128), jnp.float32),
            pltpu.VMEM((_SPR, 128), jnp.float32),
            pltpu.VMEM((2, _G, 128), jnp.float32),
            pltpu.VMEM((2, _G), jnp.int32),
            pltpu.VMEM((2, _G), jnp.int32),
            pltpu.VMEM((16,), jnp.int32),
            pltpu.SemaphoreType.DMA,
            pltpu.SemaphoreType.DMA,
        ],
        compiler_params=_sc_params,
    )
    def _reduce_p(xcat_hbm, owned_hbm, cnt_hbm, s2_hbm, mm_hbm,
                  sum_v, mm_v, gbuf, encb, srcb, cntv, gsem0, gsem1):
        wid = _wid()
        gsem = (gsem0, gsem1)
        obase = wid * _NPT

        pltpu.sync_copy(cnt_hbm.at[wid, 0], cntv)
        cnt = cntv[pl.ds(0, 16)][0]
        nch = (cnt + _G - 1) // _G

        ninf = jnp.full((16,), -jnp.inf, jnp.float32)
        pinf = jnp.full((16,), jnp.inf, jnp.float32)
        zero16 = jnp.zeros((16,), jnp.float32)

        def gather_cp(slot):
            return pltpu.make_async_copy(xcat_hbm.at[srcb.at[slot]],
                                         gbuf.at[slot], gsem[slot])

        def initb(i, carry):
            for k in range(4):
                sum_v[i, pl.ds(k * 16, 16)] = zero16
                sum_v[i, pl.ds(64 + k * 16, 16)] = zero16
                mm_v[i, pl.ds(k * 16, 16)] = ninf
                mm_v[i, pl.ds(64 + k * 16, 16)] = pinf
            return carry

        lax.fori_loop(0, _SPR, initb, 0)

        def prep(g, slot):
            pltpu.sync_copy(owned_hbm.at[wid, 0, pl.ds(g * _G, _G)],
                            encb.at[slot])
            for q in range(_G // 16):
                enc = encb[slot, pl.ds(q * 16, 16)]
                srcb[slot, pl.ds(q * 16, 16)] = (
                    jnp.right_shift(enc, 9) * 2 + p)

        @pl.when(nch >= 1)
        def _():
            prep(0, 0)
            gather_cp(0).start()

        def outer(gg, carry):
            for b in range(2):
                g = gg * 2 + b
                slot, nxt = b, 1 - b

                @pl.when(g < nch)
                def _():
                    @pl.when(g + 1 < nch)
                    def _():
                        prep(g + 1, nxt)
                        gather_cp(nxt).start()

                    gather_cp(slot).wait()

                    def merge(cur, regs):
                        ss, qq, xx, nn = regs
                        for k in range(4):
                            cs = pl.ds(k * 16, 16)
                            c2 = pl.ds(64 + k * 16, 16)
                            sum_v[cur, cs] = sum_v[cur, cs] + ss[k]
                            sum_v[cur, c2] = sum_v[cur, c2] + qq[k]
                            mm_v[cur, cs] = jnp.maximum(mm_v[cur, cs], xx[k])
                            mm_v[cur, c2] = jnp.minimum(mm_v[cur, c2], nn[k])

                    neutral = ((zero16,) * 4, (zero16,) * 4,
                               (ninf,) * 4, (pinf,) * 4)

                    def rmw(q, carry):
                        cur, ss, qq, xx, nn = carry
                        ss, qq, xx, nn = (list(ss), list(qq),
                                          list(xx), list(nn))
                        enc = encb[slot, pl.ds(q * 16, 16)]
                        for j in range(16):
                            d = jnp.bitwise_and(enc[j], 511)
                            row = q * 16 + j
                            fl = d != cur

                            @pl.when(fl)
                            def _():
                                merge(cur, (ss, qq, xx, nn))

                            for k in range(4):
                                cs = pl.ds(k * 16, 16)
                                c2 = pl.ds(64 + k * 16, 16)
                                m = gbuf[slot, row, cs]
                                m2 = gbuf[slot, row, c2]
                                ss[k] = jnp.where(fl, m, ss[k] + m)
                                qq[k] = jnp.where(fl, m2, qq[k] + m2)
                                xx[k] = jnp.where(
                                    fl, m, jnp.maximum(xx[k], m))
                                nn[k] = jnp.where(
                                    fl, m, jnp.minimum(nn[k], m))
                            cur = d
                        return (cur, tuple(ss), tuple(qq),
                                tuple(xx), tuple(nn))

                    fcur, fss, fqq, fxx, fnn = lax.fori_loop(
                        0, _G // 16, rmw,
                        (jnp.int32(_PAD_ENC),) + neutral)
                    merge(fcur, (fss, fqq, fxx, fnn))

            return carry

        lax.fori_loop(0, (nch + 1) // 2, outer, 0)

        pltpu.sync_copy(sum_v.at[pl.ds(0, _NPT)],
                        s2_hbm.at[pl.ds(obase, _NPT)])
        pltpu.sync_copy(mm_v.at[pl.ds(0, _NPT)],
                        mm_hbm.at[pl.ds(obase, _NPT)])

    return _reduce_p


_reduce_half = (_make_reduce(0), _make_reduce(1))


_BLK = 1000
_GRID = N_NODES // _BLK


def _prep_body(x_ref, o_ref):
    x = x_ref[...]
    for p in range(2):
        xh = x[:, 64 * p:64 * p + 64]
        o_ref[:, p, 0:64] = xh
        o_ref[:, p, 64:128] = xh * xh


@jax.jit
def _prep(x):
    return pl.pallas_call(
        _prep_body,
        grid=(_GRID,),
        in_specs=[pl.BlockSpec((_BLK, HID), lambda i: (i, 0))],
        out_specs=pl.BlockSpec((_BLK, 2, 128), lambda i: (i, 0, 0)),
        out_shape=jax.ShapeDtypeStruct((N_NODES, 2, 128), jnp.float32),
    )(x)


def _dense_body(final, s_ref, ssq_ref, mx_ref, mn_ref, deg_ref, h_ref,
                wa_ref, wb_ref, wc_ref, cb_ref, bn_ref, fcw_ref, fcb_ref,
                wih_ref, whh_ref, bih_ref, bhh_ref, lw_ref, lb_ref,
                hcat_ref, out_ref):
    deg = deg_ref[...]
    degc = jnp.maximum(deg, 1.0)
    inv = 1.0 / degc
    mean = s_ref[...] * inv
    var = jnp.maximum(ssq_ref[...] * inv - mean * mean, 0.0)
    std = jnp.sqrt(var + 1e-5)
    has = deg > 0.0
    mx = jnp.where(has, mx_ref[...], 0.0)
    mn = jnp.where(has, mn_ref[...], 0.0)
    agg = jnp.concatenate([mean, mn, mx, std], axis=1)
    logd = jnp.log(degc + 1.0)
    s1 = logd * (1.0 / _AVG_LOG)
    s2 = _AVG_LOG / logd
    x = (jnp.dot(agg, wa_ref[...], preferred_element_type=jnp.float32)
         + jnp.dot(agg * s1, wb_ref[...], preferred_element_type=jnp.float32)
         + jnp.dot(agg * s2, wc_ref[...], preferred_element_type=jnp.float32)
         + cb_ref[...])
    x = x * bn_ref[0:1, :] + bn_ref[1:2, :]
    x = jnp.dot(x, fcw_ref[...], preferred_element_type=jnp.float32) + fcb_ref[...]
    x = jnp.maximum(x, 0.0)
    h = h_ref[...]
    gi = jnp.dot(x, wih_ref[...], preferred_element_type=jnp.float32) + bih_ref[...]
    gh = jnp.dot(h, whh_ref[...], preferred_element_type=jnp.float32) + bhh_ref[...]
    i_r, i_z, i_n = gi[:, :HID], gi[:, HID:2 * HID], gi[:, 2 * HID:]
    h_r, h_z, h_n = gh[:, :HID], gh[:, HID:2 * HID], gh[:, 2 * HID:]
    r = jax.nn.sigmoid(i_r + h_r)
    z = jax.nn.sigmoid(i_z + h_z)
    ng = jnp.tanh(i_n + r * h_n)
    hnew = (1.0 - z) * ng + z * h
    for p in range(2):
        hh = hnew[:, 64 * p:64 * p + 64]
        hcat_ref[:, p, 0:64] = hh
        hcat_ref[:, p, 64:128] = hh * hh
    if final:
        out_ref[...] = (jnp.dot(hnew, lw_ref[...],
                                preferred_element_type=jnp.float32) + lb_ref[...])


def _node_spec(cols):
    return pl.BlockSpec((_BLK, cols), lambda i: (i, 0))


def _rep_spec(shape):
    nd = len(shape)
    return pl.BlockSpec(shape, lambda i: (0,) * nd)


@functools.partial(jax.jit, static_argnames=("final",))
def _dense_layer(s, ssq, mx, mn, deg, h, wa, wb, wc, cb, bn, fcw, fcb,
                 wih, whh, bih, bhh, lw, lb, final=False):
    out_shapes = [jax.ShapeDtypeStruct((N_NODES, 2, 128), jnp.float32),
                  jax.ShapeDtypeStruct((N_NODES, HID), jnp.float32)]
    in_specs = [_node_spec(HID)] * 4 + [_node_spec(1), _node_spec(HID)]
    in_specs += [_rep_spec(w.shape) for w in
                 (wa, wb, wc, cb, bn, fcw, fcb, wih, whh, bih, bhh, lw, lb)]
    return pl.pallas_call(
        functools.partial(_dense_body, final),
        grid=(_GRID,),
        in_specs=in_specs,
        out_specs=[pl.BlockSpec((_BLK, 2, 128), lambda i: (i, 0, 0)),
                   _node_spec(HID)],
        out_shape=out_shapes,
    )(s, ssq, mx, mn, deg, h, wa, wb, wc, cb, bn, fcw, fcb,
      wih, whh, bih, bhh, lw, lb)


def kernel(x, edge_index, batch, params):
    src = edge_index[0]
    dst = edge_index[1]
    owned, cnt2, degw = _route(src, dst)
    deg = degw[:, 0, :_NPT].reshape(-1)[:N_NODES, None]

    xcat = jnp.pad(_prep(x).reshape(2 * N_NODES, 128),
                   ((0, 2 * _NPAD - 2 * N_NODES), (0, 0)))
    h = jnp.zeros((N_NODES, HID), jnp.float32)
    bih = params['gru_b_ih'][None, :]
    bhh = params['gru_b_hh'][None, :]
    wih = params['gru_w_ih'].T
    whh = params['gru_w_hh'].T
    lw = params['last_w'].T
    lb = params['last_b'][None, :]
    out = None
    for i in range(NUM_LAYERS):
        halves = [_reduce_half[p](xcat, owned, cnt2) for p in range(2)]
        sfull = jnp.concatenate([halves[p][0][:N_NODES, :64]
                                 for p in range(2)], 1)
        ssq = jnp.concatenate([halves[p][0][:N_NODES, 64:]
                               for p in range(2)], 1)
        mx = jnp.concatenate([halves[p][1][:N_NODES, :64]
                              for p in range(2)], 1)
        mn = jnp.concatenate([halves[p][1][:N_NODES, 64:]
                              for p in range(2)], 1)
        w = params['conv%d_w' % i]
        wa = w[:, :512].T
        wb = w[:, 512:1024].T
        wc = w[:, 1024:].T
        cb = params['conv%d_b' % i][None, :]
        g = params['bn%d_g' % i] / math.sqrt(1.0 + 1e-5)
        bn = jnp.stack([g, params['bn%d_b' % i]], axis=0)
        fcw = params['fc%d_w' % i].T
        fcb = params['fc%d_b' % i][None, :]
        final = i == NUM_LAYERS - 1
        hcat, out = _dense_layer(
            sfull, ssq, mx, mn, deg, h, wa, wb, wc, cb, bn, fcw, fcb,
            wih, whh, bih, bhh, lw, lb, final=final)
        h = jnp.concatenate([hcat[:, p, :64] for p in range(2)], 1)
        if not final:
            xcat = jnp.pad(hcat.reshape(2 * N_NODES, 128),
                           ((0, 2 * _NPAD - 2 * N_NODES), (0, 0)))
    return out


# speedup vs baseline: 4.8376x; 1.0739x over previous
import functools
import math

import jax
import jax.numpy as jnp
from jax import lax
from jax.experimental import pallas as pl
from jax.experimental.pallas import tpu as pltpu
from jax.experimental.pallas import tpu_sc as plsc

N_NODES = 10000
N_EDGES = 320000
HID = 128
NUM_LAYERS = 3
_AVG_LOG = math.log(33.0)

_NT = 32
_NPT = 320
_NPAD = _NT * _NPT
_SPR = 328
_CAP = 11264
_PAD_ENC = _NPT
_CH = 3200
_G = 64

_sc_mesh = plsc.VectorSubcoreMesh(core_axis_name="c", subcore_axis_name="s")
_sc_params = pltpu.CompilerParams(needs_layout_passes=False)


def _wid():
    return lax.axis_index("c") * 16 + lax.axis_index("s")


@functools.partial(
    pl.kernel,
    out_type=[
        jax.ShapeDtypeStruct((_NT, 1, _CAP), jnp.int32),
        jax.ShapeDtypeStruct((_NT, 1, 16), jnp.int32),
        jax.ShapeDtypeStruct((_NT, 1, 336), jnp.float32),
    ],
    mesh=_sc_mesh,
    scratch_types=[
        pltpu.VMEM((_CAP + 16,), jnp.int32),
        pltpu.VMEM((_CAP + 16,), jnp.int32),
        pltpu.VMEM((_CH,), jnp.int32),
        pltpu.VMEM((_CH,), jnp.int32),
        pltpu.VMEM((336,), jnp.float32),
        pltpu.VMEM((336,), jnp.int32),
        pltpu.VMEM((16,), jnp.int32),
        pltpu.VMEM((16,), jnp.int32),
    ],
    compiler_params=_sc_params,
)
def _route(src_hbm, dst_hbm, owned_hbm, cnt_hbm, deg_hbm,
           owned_v, sorted_v, src_v, dst_v, deg_v, off_v, dlbuf, misc_v):
    wid = _wid()
    lo = wid * _NPT
    pad16 = jnp.full((16,), _PAD_ENC, jnp.int32)

    def initb(i, carry):
        owned_v[pl.ds(i * 16, 16)] = pad16
        sorted_v[pl.ds(i * 16, 16)] = pad16
        return carry

    lax.fori_loop(0, (_CAP + 16) // 16, initb, 0)
    zero16 = jnp.zeros((16,), jnp.float32)
    for i in range(21):
        deg_v[pl.ds(i * 16, 16)] = zero16

    def chunk_body(g, cnt):
        pltpu.sync_copy(src_hbm.at[pl.ds(g * _CH, _CH)], src_v)
        pltpu.sync_copy(dst_hbm.at[pl.ds(g * _CH, _CH)], dst_v)

        def scan16(i, cnt):
            dv = dst_v[pl.ds(i * 16, 16)]
            sv = src_v[pl.ds(i * 16, 16)]
            dl = dv - lo
            m = jnp.logical_and(dl >= 0, dl < _NPT)
            enc = sv * 512 + dl
            plsc.store_compressed(owned_v.at[pl.ds(cnt, 16)], enc, mask=m)
            return cnt + plsc.all_reduce_population_count(m)[0]

        return lax.fori_loop(0, _CH // 16, scan16, cnt)

    cnt = lax.fori_loop(0, N_EDGES // _CH, chunk_body, jnp.int32(0))

    ones16 = jnp.ones((16,), jnp.float32)

    def degb(i, carry):
        enc = owned_v[pl.ds(i * 16, 16)]
        dl = jnp.bitwise_and(enc, 511)
        plsc.addupdate_scatter(deg_v, [dl], ones16)
        return carry

    lax.fori_loop(0, (cnt + 15) // 16, degb, 0)

    iota = lax.iota(jnp.int32, 16)
    carry = jnp.int32(0)
    for i in range(21):
        v = deg_v[pl.ds(i * 16, 16)].astype(jnp.int32)
        cum = plsc.cumsum(v)
        off_v[pl.ds(i * 16, 16)] = cum - v + carry
        carry = carry + cum[15]

    ones16i = jnp.ones((16,), jnp.int32)

    def placeb(i, carry):
        enc = owned_v[pl.ds(i * 16, 16)]
        dl = jnp.bitwise_and(enc, 511)
        dlbuf[pl.ds(0, 16)] = dl
        rank = jnp.zeros((16,), jnp.int32)
        for sft in range(1, 16):
            msk = iota >= sft
            sh = plsc.load_gather(dlbuf, [iota - sft], mask=msk)
            eq = jnp.logical_and(sh == dl, msk)
            rank = rank + eq.astype(jnp.int32)
        pos = plsc.load_gather(off_v, [dl]) + rank
        plsc.store_scatter(sorted_v, [pos], enc)
        plsc.addupdate_scatter(off_v, [dl], ones16i)
        return carry

    lax.fori_loop(0, (cnt + 15) // 16, placeb, 0)

    misc_v[pl.ds(0, 16)] = jnp.full((16,), cnt, jnp.int32)
    pltpu.sync_copy(sorted_v.at[pl.ds(0, _CAP)], owned_hbm.at[wid, 0])
    pltpu.sync_copy(misc_v.at[pl.ds(0, 16)], cnt_hbm.at[wid, 0])
    pltpu.sync_copy(deg_v, deg_hbm.at[wid, 0])


def _make_reduce(p):
    @functools.partial(
        pl.kernel,
        out_type=[
            jax.ShapeDtypeStruct((_NPAD, 128), jnp.float32),
            jax.ShapeDtypeStruct((_NPAD, 128), jnp.float32),
        ],
        mesh=_sc_mesh,
        scratch_types=[
            pltpu.VMEM((_SPR, 128), jnp.float32),
            pltpu.VMEM((_SPR, 128), jnp.float32),
            pltpu.VMEM((2, _G, 128), jnp.float32),
            pltpu.VMEM((2, _G), jnp.int32),
            pltpu.VMEM((2, _G), jnp.int32),
            pltpu.VMEM((16,), jnp.int32),
            pltpu.SemaphoreType.DMA,
            pltpu.SemaphoreType.DMA,
        ],
        compiler_params=_sc_params,
    )
    def _reduce_p(xcat_hbm, owned_hbm, cnt_hbm, s2_hbm, mm_hbm,
                  sum_v, mm_v, gbuf, encb, srcb, cntv, gsem0, gsem1):
        wid = _wid()
        gsem = (gsem0, gsem1)
        obase = wid * _NPT

        pltpu.sync_copy(cnt_hbm.at[wid, 0], cntv)
        cnt = cntv[pl.ds(0, 16)][0]
        nch = (cnt + _G - 1) // _G

        ninf = jnp.full((16,), -jnp.inf, jnp.float32)
        pinf = jnp.full((16,), jnp.inf, jnp.float32)
        zero16 = jnp.zeros((16,), jnp.float32)

        def gather_cp(slot):
            return pltpu.make_async_copy(xcat_hbm.at[srcb.at[slot]],
                                         gbuf.at[slot], gsem[slot])

        def initb(i, carry):
            for k in range(4):
                sum_v[i, pl.ds(k * 16, 16)] = zero16
                sum_v[i, pl.ds(64 + k * 16, 16)] = zero16
                mm_v[i, pl.ds(k * 16, 16)] = ninf
                mm_v[i, pl.ds(64 + k * 16, 16)] = pinf
            return carry

        lax.fori_loop(0, _SPR, initb, 0)

        def prep(g, slot):
            pltpu.sync_copy(owned_hbm.at[wid, 0, pl.ds(g * _G, _G)],
                            encb.at[slot])
            for q in range(_G // 16):
                enc = encb[slot, pl.ds(q * 16, 16)]
                srcb[slot, pl.ds(q * 16, 16)] = jnp.right_shift(enc, 9)

        @pl.when(nch >= 1)
        def _():
            prep(0, 0)
            gather_cp(0).start()

        def outer(gg, carry):
            for b in range(2):
                g = gg * 2 + b
                slot, nxt = b, 1 - b

                @pl.when(g < nch)
                def _():
                    @pl.when(g + 1 < nch)
                    def _():
                        prep(g + 1, nxt)
                        gather_cp(nxt).start()

                    gather_cp(slot).wait()

                    def merge(cur, regs):
                        ss, qq, xx, nn = regs
                        for k in range(4):
                            cs = pl.ds(k * 16, 16)
                            c2 = pl.ds(64 + k * 16, 16)
                            sum_v[cur, cs] = sum_v[cur, cs] + ss[k]
                            sum_v[cur, c2] = sum_v[cur, c2] + qq[k]
                            mm_v[cur, cs] = jnp.maximum(mm_v[cur, cs], xx[k])
                            mm_v[cur, c2] = jnp.minimum(mm_v[cur, c2], nn[k])

                    neutral = ((zero16,) * 4, (zero16,) * 4,
                               (ninf,) * 4, (pinf,) * 4)

                    def rmw(q, carry):
                        cur, ss, qq, xx, nn = carry
                        ss, qq, xx, nn = (list(ss), list(qq),
                                          list(xx), list(nn))
                        enc = encb[slot, pl.ds(q * 16, 16)]
                        for j in range(16):
                            d = jnp.bitwise_and(enc[j], 511)
                            row = q * 16 + j
                            fl = d != cur

                            @pl.when(fl)
                            def _():
                                merge(cur, (ss, qq, xx, nn))

                            for k in range(4):
                                cg = pl.ds(64 * p + k * 16, 16)
                                m = gbuf[slot, row, cg]
                                m2 = m * m
                                ss[k] = jnp.where(fl, m, ss[k] + m)
                                qq[k] = jnp.where(fl, m2, qq[k] + m2)
                                xx[k] = jnp.where(
                                    fl, m, jnp.maximum(xx[k], m))
                                nn[k] = jnp.where(
                                    fl, m, jnp.minimum(nn[k], m))
                            cur = d
                        return (cur, tuple(ss), tuple(qq),
                                tuple(xx), tuple(nn))

                    fcur, fss, fqq, fxx, fnn = lax.fori_loop(
                        0, _G // 16, rmw,
                        (jnp.int32(_PAD_ENC),) + neutral)
                    merge(fcur, (fss, fqq, fxx, fnn))

            return carry

        lax.fori_loop(0, (nch + 1) // 2, outer, 0)

        pltpu.sync_copy(sum_v.at[pl.ds(0, _NPT)],
                        s2_hbm.at[pl.ds(obase, _NPT)])
        pltpu.sync_copy(mm_v.at[pl.ds(0, _NPT)],
                        mm_hbm.at[pl.ds(obase, _NPT)])

    return _reduce_p


_reduce_half = (_make_reduce(0), _make_reduce(1))


_BLK = 1000
_GRID = N_NODES // _BLK


def _dense_body(final, s_ref, ssq_ref, mx_ref, mn_ref, deg_ref, h_ref,
                wa_ref, wb_ref, wc_ref, cb_ref, bn_ref, fcw_ref, fcb_ref,
                wih_ref, whh_ref, bih_ref, bhh_ref, lw_ref, lb_ref,
                hcat_ref, out_ref):
    deg = deg_ref[...]
    degc = jnp.maximum(deg, 1.0)
    inv = 1.0 / degc
    mean = s_ref[...] * inv
    var = jnp.maximum(ssq_ref[...] * inv - mean * mean, 0.0)
    std = jnp.sqrt(var + 1e-5)
    has = deg > 0.0
    mx = jnp.where(has, mx_ref[...], 0.0)
    mn = jnp.where(has, mn_ref[...], 0.0)
    agg = jnp.concatenate([mean, mn, mx, std], axis=1)
    logd = jnp.log(degc + 1.0)
    s1 = logd * (1.0 / _AVG_LOG)
    s2 = _AVG_LOG / logd
    x = (jnp.dot(agg, wa_ref[...], preferred_element_type=jnp.float32)
         + jnp.dot(agg * s1, wb_ref[...], preferred_element_type=jnp.float32)
         + jnp.dot(agg * s2, wc_ref[...], preferred_element_type=jnp.float32)
         + cb_ref[...])
    x = x * bn_ref[0:1, :] + bn_ref[1:2, :]
    x = jnp.dot(x, fcw_ref[...], preferred_element_type=jnp.float32) + fcb_ref[...]
    x = jnp.maximum(x, 0.0)
    h = h_ref[...]
    gi = jnp.dot(x, wih_ref[...], preferred_element_type=jnp.float32) + bih_ref[...]
    gh = jnp.dot(h, whh_ref[...], preferred_element_type=jnp.float32) + bhh_ref[...]
    i_r, i_z, i_n = gi[:, :HID], gi[:, HID:2 * HID], gi[:, 2 * HID:]
    h_r, h_z, h_n = gh[:, :HID], gh[:, HID:2 * HID], gh[:, 2 * HID:]
    r = jax.nn.sigmoid(i_r + h_r)
    z = jax.nn.sigmoid(i_z + h_z)
    ng = jnp.tanh(i_n + r * h_n)
    hnew = (1.0 - z) * ng + z * h
    hcat_ref[...] = hnew
    if final:
        out_ref[...] = (jnp.dot(hnew, lw_ref[...],
                                preferred_element_type=jnp.float32) + lb_ref[...])


def _node_spec(cols):
    return pl.BlockSpec((_BLK, cols), lambda i: (i, 0))


def _rep_spec(shape):
    nd = len(shape)
    return pl.BlockSpec(shape, lambda i: (0,) * nd)


@functools.partial(jax.jit, static_argnames=("final",))
def _dense_layer(s, ssq, mx, mn, deg, h, wa, wb, wc, cb, bn, fcw, fcb,
                 wih, whh, bih, bhh, lw, lb, final=False):
    out_shapes = [jax.ShapeDtypeStruct((N_NODES, HID), jnp.float32),
                  jax.ShapeDtypeStruct((N_NODES, HID), jnp.float32)]
    in_specs = [_node_spec(HID)] * 4 + [_node_spec(1), _node_spec(HID)]
    in_specs += [_rep_spec(w.shape) for w in
                 (wa, wb, wc, cb, bn, fcw, fcb, wih, whh, bih, bhh, lw, lb)]
    return pl.pallas_call(
        functools.partial(_dense_body, final),
        grid=(_GRID,),
        in_specs=in_specs,
        out_specs=[_node_spec(HID), _node_spec(HID)],
        out_shape=out_shapes,
    )(s, ssq, mx, mn, deg, h, wa, wb, wc, cb, bn, fcw, fcb,
      wih, whh, bih, bhh, lw, lb)


def kernel(x, edge_index, batch, params):
    src = edge_index[0]
    dst = edge_index[1]
    owned, cnt2, degw = _route(src, dst)
    deg = degw[:, 0, :_NPT].reshape(-1)[:N_NODES, None]

    xcat = jnp.pad(x, ((0, _NPAD - N_NODES), (0, 0)))
    h = jnp.zeros((N_NODES, HID), jnp.float32)
    bih = params['gru_b_ih'][None, :]
    bhh = params['gru_b_hh'][None, :]
    wih = params['gru_w_ih'].T
    whh = params['gru_w_hh'].T
    lw = params['last_w'].T
    lb = params['last_b'][None, :]
    out = None
    for i in range(NUM_LAYERS):
        halves = [_reduce_half[p](xcat, owned, cnt2) for p in range(2)]
        sfull = jnp.concatenate([halves[p][0][:N_NODES, :64]
                                 for p in range(2)], 1)
        ssq = jnp.concatenate([halves[p][0][:N_NODES, 64:]
                               for p in range(2)], 1)
        mx = jnp.concatenate([halves[p][1][:N_NODES, :64]
                              for p in range(2)], 1)
        mn = jnp.concatenate([halves[p][1][:N_NODES, 64:]
                              for p in range(2)], 1)
        w = params['conv%d_w' % i]
        wa = w[:, :512].T
        wb = w[:, 512:1024].T
        wc = w[:, 1024:].T
        cb = params['conv%d_b' % i][None, :]
        g = params['bn%d_g' % i] / math.sqrt(1.0 + 1e-5)
        bn = jnp.stack([g, params['bn%d_b' % i]], axis=0)
        fcw = params['fc%d_w' % i].T
        fcb = params['fc%d_b' % i][None, :]
        final = i == NUM_LAYERS - 1
        hcat, out = _dense_layer(
            sfull, ssq, mx, mn, deg, h, wa, wb, wc, cb, bn, fcw, fcb,
            wih, whh, bih, bhh, lw, lb, final=final)
        h = hcat
        if not final:
            xcat = jnp.pad(h, ((0, _NPAD - N_NODES), (0, 0)))
    return out


# speedup vs baseline: 5.1002x; 1.0543x over previous
import functools
import math

import jax
import jax.numpy as jnp
from jax import lax
from jax.experimental import pallas as pl
from jax.experimental.pallas import tpu as pltpu
from jax.experimental.pallas import tpu_sc as plsc

N_NODES = 10000
N_EDGES = 320000
HID = 128
NUM_LAYERS = 3
_AVG_LOG = math.log(33.0)

_NT = 32
_NPT = 320
_NPAD = _NT * _NPT
_SPR = 328
_CAP = 11264
_PAD_ENC = _NPT
_CH = 3200
_G = 64

_sc_mesh = plsc.VectorSubcoreMesh(core_axis_name="c", subcore_axis_name="s")
_sc_params = pltpu.CompilerParams(needs_layout_passes=False)


def _wid():
    return lax.axis_index("c") * 16 + lax.axis_index("s")


@functools.partial(
    pl.kernel,
    out_type=[
        jax.ShapeDtypeStruct((_NT, 1, _CAP), jnp.int32),
        jax.ShapeDtypeStruct((_NT, 1, 16), jnp.int32),
        jax.ShapeDtypeStruct((_NT, 1, 336), jnp.float32),
    ],
    mesh=_sc_mesh,
    scratch_types=[
        pltpu.VMEM((_CAP + 16,), jnp.int32),
        pltpu.VMEM((_CAP + 16,), jnp.int32),
        pltpu.VMEM((2, _CH), jnp.int32),
        pltpu.VMEM((2, _CH), jnp.int32),
        pltpu.VMEM((336,), jnp.float32),
        pltpu.VMEM((336,), jnp.int32),
        pltpu.VMEM((16,), jnp.int32),
        pltpu.VMEM((16,), jnp.int32),
        pltpu.SemaphoreType.DMA,
        pltpu.SemaphoreType.DMA,
        pltpu.SemaphoreType.DMA,
        pltpu.SemaphoreType.DMA,
    ],
    compiler_params=_sc_params,
)
def _route(src_hbm, dst_hbm, owned_hbm, cnt_hbm, deg_hbm,
           owned_v, sorted_v, src_v, dst_v, deg_v, off_v, dlbuf, misc_v,
           ss0, ss1, sd0, sd1):
    wid = _wid()
    lo = wid * _NPT
    pad16 = jnp.full((16,), _PAD_ENC, jnp.int32)

    def initb(i, carry):
        owned_v[pl.ds(i * 16, 16)] = pad16
        sorted_v[pl.ds(i * 16, 16)] = pad16
        return carry

    lax.fori_loop(0, (_CAP + 16) // 16, initb, 0)
    zero16 = jnp.zeros((16,), jnp.float32)
    for i in range(21):
        deg_v[pl.ds(i * 16, 16)] = zero16

    ssem = (ss0, ss1)
    dsem = (sd0, sd1)

    def src_cp(g, slot):
        return pltpu.make_async_copy(src_hbm.at[pl.ds(g * _CH, _CH)],
                                     src_v.at[slot], ssem[slot])

    def dst_cp(g, slot):
        return pltpu.make_async_copy(dst_hbm.at[pl.ds(g * _CH, _CH)],
                                     dst_v.at[slot], dsem[slot])

    nch_r = N_EDGES // _CH
    src_cp(0, 0).start()
    dst_cp(0, 0).start()
    cnt = jnp.int32(0)
    for g in range(nch_r):
        slot = g & 1
        if g + 1 < nch_r:
            src_cp(g + 1, 1 - slot).start()
            dst_cp(g + 1, 1 - slot).start()
        src_cp(g, slot).wait()
        dst_cp(g, slot).wait()

        def scan16(i, cnt):
            dv = dst_v[slot, pl.ds(i * 16, 16)]
            sv = src_v[slot, pl.ds(i * 16, 16)]
            dl = dv - lo
            m = jnp.logical_and(dl >= 0, dl < _NPT)
            enc = sv * 512 + dl
            plsc.store_compressed(owned_v.at[pl.ds(cnt, 16)], enc, mask=m)
            return cnt + plsc.all_reduce_population_count(m)[0]

        cnt = lax.fori_loop(0, _CH // 16, scan16, cnt)

    ones16 = jnp.ones((16,), jnp.float32)

    def degb(i, carry):
        enc = owned_v[pl.ds(i * 16, 16)]
        dl = jnp.bitwise_and(enc, 511)
        plsc.addupdate_scatter(deg_v, [dl], ones16)
        return carry

    lax.fori_loop(0, (cnt + 15) // 16, degb, 0)

    iota = lax.iota(jnp.int32, 16)
    carry = jnp.int32(0)
    for i in range(21):
        v = deg_v[pl.ds(i * 16, 16)].astype(jnp.int32)
        cum = plsc.cumsum(v)
        off_v[pl.ds(i * 16, 16)] = cum - v + carry
        carry = carry + cum[15]

    ones16i = jnp.ones((16,), jnp.int32)

    def placeb(i, carry):
        enc = owned_v[pl.ds(i * 16, 16)]
        dl = jnp.bitwise_and(enc, 511)
        dlbuf[pl.ds(0, 16)] = dl
        rank = jnp.zeros((16,), jnp.int32)
        for sft in range(1, 16):
            msk = iota >= sft
            sh = plsc.load_gather(dlbuf, [iota - sft], mask=msk)
            eq = jnp.logical_and(sh == dl, msk)
            rank = rank + eq.astype(jnp.int32)
        pos = plsc.load_gather(off_v, [dl]) + rank
        plsc.store_scatter(sorted_v, [pos], enc)
        plsc.addupdate_scatter(off_v, [dl], ones16i)
        return carry

    lax.fori_loop(0, (cnt + 15) // 16, placeb, 0)

    misc_v[pl.ds(0, 16)] = jnp.full((16,), cnt, jnp.int32)
    pltpu.sync_copy(sorted_v.at[pl.ds(0, _CAP)], owned_hbm.at[wid, 0])
    pltpu.sync_copy(misc_v.at[pl.ds(0, 16)], cnt_hbm.at[wid, 0])
    pltpu.sync_copy(deg_v, deg_hbm.at[wid, 0])


def _make_reduce(p):
    @functools.partial(
        pl.kernel,
        out_type=[
            jax.ShapeDtypeStruct((_NPAD, 128), jnp.float32),
            jax.ShapeDtypeStruct((_NPAD, 128), jnp.float32),
        ],
        mesh=_sc_mesh,
        scratch_types=[
            pltpu.VMEM((_SPR, 128), jnp.float32),
            pltpu.VMEM((_SPR, 128), jnp.float32),
            pltpu.VMEM((2, _G, 128), jnp.float32),
            pltpu.VMEM((2, _G), jnp.int32),
            pltpu.VMEM((2, _G), jnp.int32),
            pltpu.VMEM((16,), jnp.int32),
            pltpu.SemaphoreType.DMA,
            pltpu.SemaphoreType.DMA,
        ],
        compiler_params=_sc_params,
    )
    def _reduce_p(xcat_hbm, owned_hbm, cnt_hbm, s2_hbm, mm_hbm,
                  sum_v, mm_v, gbuf, encb, srcb, cntv, gsem0, gsem1):
        wid = _wid()
        gsem = (gsem0, gsem1)
        obase = wid * _NPT

        pltpu.sync_copy(cnt_hbm.at[wid, 0], cntv)
        cnt = cntv[pl.ds(0, 16)][0]
        nch = (cnt + _G - 1) // _G

        ninf = jnp.full((16,), -jnp.inf, jnp.float32)
        pinf = jnp.full((16,), jnp.inf, jnp.float32)
        zero16 = jnp.zeros((16,), jnp.float32)

        def gather_cp(slot):
            return pltpu.make_async_copy(xcat_hbm.at[srcb.at[slot]],
                                         gbuf.at[slot], gsem[slot])

        def initb(i, carry):
            for k in range(4):
                sum_v[i, pl.ds(k * 16, 16)] = zero16
                sum_v[i, pl.ds(64 + k * 16, 16)] = zero16
                mm_v[i, pl.ds(k * 16, 16)] = ninf
                mm_v[i, pl.ds(64 + k * 16, 16)] = pinf
            return carry

        lax.fori_loop(0, _SPR, initb, 0)

        def prep(g, slot):
            pltpu.sync_copy(owned_hbm.at[wid, 0, pl.ds(g * _G, _G)],
                            encb.at[slot])
            for q in range(_G // 16):
                enc = encb[slot, pl.ds(q * 16, 16)]
                srcb[slot, pl.ds(q * 16, 16)] = jnp.right_shift(enc, 9)

        @pl.when(nch >= 1)
        def _():
            prep(0, 0)
            gather_cp(0).start()

        def outer(gg, carry):
            for b in range(2):
                g = gg * 2 + b
                slot, nxt = b, 1 - b

                @pl.when(g < nch)
                def _():
                    @pl.when(g + 1 < nch)
                    def _():
                        prep(g + 1, nxt)
                        gather_cp(nxt).start()

                    gather_cp(slot).wait()

                    def merge(cur, regs):
                        ss, qq, xx, nn = regs
                        for k in range(4):
                            cs = pl.ds(k * 16, 16)
                            c2 = pl.ds(64 + k * 16, 16)
                            sum_v[cur, cs] = sum_v[cur, cs] + ss[k]
                            sum_v[cur, c2] = sum_v[cur, c2] + qq[k]
                            mm_v[cur, cs] = jnp.maximum(mm_v[cur, cs], xx[k])
                            mm_v[cur, c2] = jnp.minimum(mm_v[cur, c2], nn[k])

                    neutral = ((zero16,) * 4, (zero16,) * 4,
                               (ninf,) * 4, (pinf,) * 4)

                    def rmw(q, carry):
                        cur, ss, qq, xx, nn = carry
                        ss, qq, xx, nn = (list(ss), list(qq),
                                          list(xx), list(nn))
                        enc = encb[slot, pl.ds(q * 16, 16)]
                        for j in range(16):
                            d = jnp.bitwise_and(enc[j], 511)
                            row = q * 16 + j
                            fl = d != cur

                            @pl.when(fl)
                            def _():
                                merge(cur, (ss, qq, xx, nn))

                            for k in range(4):
                                cg = pl.ds(64 * p + k * 16, 16)
                                m = gbuf[slot, row, cg]
                                m2 = m * m
                                ss[k] = jnp.where(fl, m, ss[k] + m)
                                qq[k] = jnp.where(fl, m2, qq[k] + m2)
                                xx[k] = jnp.where(
                                    fl, m, jnp.maximum(xx[k], m))
                                nn[k] = jnp.where(
                                    fl, m, jnp.minimum(nn[k], m))
                            cur = d
                        return (cur, tuple(ss), tuple(qq),
                                tuple(xx), tuple(nn))

                    fcur, fss, fqq, fxx, fnn = lax.fori_loop(
                        0, _G // 16, rmw,
                        (jnp.int32(_PAD_ENC),) + neutral)
                    merge(fcur, (fss, fqq, fxx, fnn))

            return carry

        lax.fori_loop(0, (nch + 1) // 2, outer, 0)

        pltpu.sync_copy(sum_v.at[pl.ds(0, _NPT)],
                        s2_hbm.at[pl.ds(obase, _NPT)])
        pltpu.sync_copy(mm_v.at[pl.ds(0, _NPT)],
                        mm_hbm.at[pl.ds(obase, _NPT)])

    return _reduce_p


_reduce_half = (_make_reduce(0), _make_reduce(1))


_BLK = 1000
_GRID = N_NODES // _BLK


def _dense_body(final, s_ref, ssq_ref, mx_ref, mn_ref, deg_ref, h_ref,
                wa_ref, wb_ref, wc_ref, cb_ref, bn_ref, fcw_ref, fcb_ref,
                wih_ref, whh_ref, bih_ref, bhh_ref, lw_ref, lb_ref,
                hcat_ref, out_ref):
    deg = deg_ref[...]
    degc = jnp.maximum(deg, 1.0)
    inv = 1.0 / degc
    mean = s_ref[...] * inv
    var = jnp.maximum(ssq_ref[...] * inv - mean * mean, 0.0)
    std = jnp.sqrt(var + 1e-5)
    has = deg > 0.0
    mx = jnp.where(has, mx_ref[...], 0.0)
    mn = jnp.where(has, mn_ref[...], 0.0)
    agg = jnp.concatenate([mean, mn, mx, std], axis=1)
    logd = jnp.log(degc + 1.0)
    s1 = logd * (1.0 / _AVG_LOG)
    s2 = _AVG_LOG / logd
    x = (jnp.dot(agg, wa_ref[...], preferred_element_type=jnp.float32)
         + jnp.dot(agg * s1, wb_ref[...], preferred_element_type=jnp.float32)
         + jnp.dot(agg * s2, wc_ref[...], preferred_element_type=jnp.float32)
         + cb_ref[...])
    x = x * bn_ref[0:1, :] + bn_ref[1:2, :]
    x = jnp.dot(x, fcw_ref[...], preferred_element_type=jnp.float32) + fcb_ref[...]
    x = jnp.maximum(x, 0.0)
    h = h_ref[...]
    gi = jnp.dot(x, wih_ref[...], preferred_element_type=jnp.float32) + bih_ref[...]
    gh = jnp.dot(h, whh_ref[...], preferred_element_type=jnp.float32) + bhh_ref[...]
    i_r, i_z, i_n = gi[:, :HID], gi[:, HID:2 * HID], gi[:, 2 * HID:]
    h_r, h_z, h_n = gh[:, :HID], gh[:, HID:2 * HID], gh[:, 2 * HID:]
    r = jax.nn.sigmoid(i_r + h_r)
    z = jax.nn.sigmoid(i_z + h_z)
    ng = jnp.tanh(i_n + r * h_n)
    hnew = (1.0 - z) * ng + z * h
    hcat_ref[...] = hnew
    if final:
        out_ref[...] = (jnp.dot(hnew, lw_ref[...],
                                preferred_element_type=jnp.float32) + lb_ref[...])


def _node_spec(cols):
    return pl.BlockSpec((_BLK, cols), lambda i: (i, 0))


def _rep_spec(shape):
    nd = len(shape)
    return pl.BlockSpec(shape, lambda i: (0,) * nd)


@functools.partial(jax.jit, static_argnames=("final",))
def _dense_layer(s, ssq, mx, mn, deg, h, wa, wb, wc, cb, bn, fcw, fcb,
                 wih, whh, bih, bhh, lw, lb, final=False):
    out_shapes = [jax.ShapeDtypeStruct((N_NODES, HID), jnp.float32),
                  jax.ShapeDtypeStruct((N_NODES, HID), jnp.float32)]
    in_specs = [_node_spec(HID)] * 4 + [_node_spec(1), _node_spec(HID)]
    in_specs += [_rep_spec(w.shape) for w in
                 (wa, wb, wc, cb, bn, fcw, fcb, wih, whh, bih, bhh, lw, lb)]
    return pl.pallas_call(
        functools.partial(_dense_body, final),
        grid=(_GRID,),
        in_specs=in_specs,
        out_specs=[_node_spec(HID), _node_spec(HID)],
        out_shape=out_shapes,
    )(s, ssq, mx, mn, deg, h, wa, wb, wc, cb, bn, fcw, fcb,
      wih, whh, bih, bhh, lw, lb)


def kernel(x, edge_index, batch, params):
    src = edge_index[0]
    dst = edge_index[1]
    owned, cnt2, degw = _route(src, dst)
    deg = degw[:, 0, :_NPT].reshape(-1)[:N_NODES, None]

    xcat = jnp.pad(x, ((0, _NPAD - N_NODES), (0, 0)))
    h = jnp.zeros((N_NODES, HID), jnp.float32)
    bih = params['gru_b_ih'][None, :]
    bhh = params['gru_b_hh'][None, :]
    wih = params['gru_w_ih'].T
    whh = params['gru_w_hh'].T
    lw = params['last_w'].T
    lb = params['last_b'][None, :]
    out = None
    for i in range(NUM_LAYERS):
        halves = [_reduce_half[p](xcat, owned, cnt2) for p in range(2)]
        sfull = jnp.concatenate([halves[p][0][:N_NODES, :64]
                                 for p in range(2)], 1)
        ssq = jnp.concatenate([halves[p][0][:N_NODES, 64:]
                               for p in range(2)], 1)
        mx = jnp.concatenate([halves[p][1][:N_NODES, :64]
                              for p in range(2)], 1)
        mn = jnp.concatenate([halves[p][1][:N_NODES, 64:]
                              for p in range(2)], 1)
        w = params['conv%d_w' % i]
        wa = w[:, :512].T
        wb = w[:, 512:1024].T
        wc = w[:, 1024:].T
        cb = params['conv%d_b' % i][None, :]
        g = params['bn%d_g' % i] / math.sqrt(1.0 + 1e-5)
        bn = jnp.stack([g, params['bn%d_b' % i]], axis=0)
        fcw = params['fc%d_w' % i].T
        fcb = params['fc%d_b' % i][None, :]
        final = i == NUM_LAYERS - 1
        hcat, out = _dense_layer(
            sfull, ssq, mx, mn, deg, h, wa, wb, wc, cb, bn, fcw, fcb,
            wih, whh, bih, bhh, lw, lb, final=final)
        h = hcat
        if not final:
            xcat = jnp.pad(h, ((0, _NPAD - N_NODES), (0, 0)))
    return out


# speedup vs baseline: 5.2983x; 1.0388x over previous
import functools
import math

import jax
import jax.numpy as jnp
from jax import lax
from jax.experimental import pallas as pl
from jax.experimental.pallas import tpu as pltpu
from jax.experimental.pallas import tpu_sc as plsc

N_NODES = 10000
N_EDGES = 320000
HID = 128
NUM_LAYERS = 3
_AVG_LOG = math.log(33.0)

_NT = 32
_NPT = 320
_NPAD = _NT * _NPT
_SPR = 328
_CAP = 11264
_PAD_ENC = _NPT
_CH = 3200
_G = 64

_sc_mesh = plsc.VectorSubcoreMesh(core_axis_name="c", subcore_axis_name="s")
_sc_params = pltpu.CompilerParams(needs_layout_passes=False)


def _wid():
    return lax.axis_index("c") * 16 + lax.axis_index("s")


@functools.partial(
    pl.kernel,
    out_type=[
        jax.ShapeDtypeStruct((_NT, 1, _CAP), jnp.int32),
        jax.ShapeDtypeStruct((_NT, 1, 16), jnp.int32),
        jax.ShapeDtypeStruct((_NT, 1, 336), jnp.float32),
    ],
    mesh=_sc_mesh,
    scratch_types=[
        pltpu.VMEM((_CAP + 16,), jnp.int32),
        pltpu.VMEM((_CAP + 16,), jnp.int32),
        pltpu.VMEM((2, _CH), jnp.int32),
        pltpu.VMEM((2, _CH), jnp.int32),
        pltpu.VMEM((336,), jnp.float32),
        pltpu.VMEM((336,), jnp.int32),
        pltpu.VMEM((16,), jnp.int32),
        pltpu.VMEM((16,), jnp.int32),
        pltpu.SemaphoreType.DMA,
        pltpu.SemaphoreType.DMA,
        pltpu.SemaphoreType.DMA,
        pltpu.SemaphoreType.DMA,
    ],
    compiler_params=_sc_params,
)
def _route(src_hbm, dst_hbm, owned_hbm, cnt_hbm, deg_hbm,
           owned_v, sorted_v, src_v, dst_v, deg_v, off_v, dlbuf, misc_v,
           ss0, ss1, sd0, sd1):
    wid = _wid()
    lo = wid * _NPT
    pad16 = jnp.full((16,), _PAD_ENC, jnp.int32)

    def initb(i, carry):
        owned_v[pl.ds(i * 16, 16)] = pad16
        sorted_v[pl.ds(i * 16, 16)] = pad16
        return carry

    lax.fori_loop(0, (_CAP + 16) // 16, initb, 0)
    zero16 = jnp.zeros((16,), jnp.float32)
    for i in range(21):
        deg_v[pl.ds(i * 16, 16)] = zero16

    ssem = (ss0, ss1)
    dsem = (sd0, sd1)

    def src_cp(g, slot):
        return pltpu.make_async_copy(src_hbm.at[pl.ds(g * _CH, _CH)],
                                     src_v.at[slot], ssem[slot])

    def dst_cp(g, slot):
        return pltpu.make_async_copy(dst_hbm.at[pl.ds(g * _CH, _CH)],
                                     dst_v.at[slot], dsem[slot])

    nch_r = N_EDGES // _CH
    src_cp(0, 0).start()
    dst_cp(0, 0).start()
    cnt = jnp.int32(0)
    for g in range(nch_r):
        slot = g & 1
        if g + 1 < nch_r:
            src_cp(g + 1, 1 - slot).start()
            dst_cp(g + 1, 1 - slot).start()
        src_cp(g, slot).wait()
        dst_cp(g, slot).wait()

        def scan16(i, cnt):
            dv = dst_v[slot, pl.ds(i * 16, 16)]
            sv = src_v[slot, pl.ds(i * 16, 16)]
            dl = dv - lo
            m = jnp.logical_and(dl >= 0, dl < _NPT)
            enc = sv * 512 + dl
            plsc.store_compressed(owned_v.at[pl.ds(cnt, 16)], enc, mask=m)
            return cnt + plsc.all_reduce_population_count(m)[0]

        cnt = lax.fori_loop(0, _CH // 16, scan16, cnt)

    ones16 = jnp.ones((16,), jnp.float32)

    def degb(i, carry):
        enc = owned_v[pl.ds(i * 16, 16)]
        dl = jnp.bitwise_and(enc, 511)
        plsc.addupdate_scatter(deg_v, [dl], ones16)
        return carry

    lax.fori_loop(0, (cnt + 15) // 16, degb, 0)

    iota = lax.iota(jnp.int32, 16)
    carry = jnp.int32(0)
    for i in range(21):
        v = deg_v[pl.ds(i * 16, 16)].astype(jnp.int32)
        cum = plsc.cumsum(v)
        off_v[pl.ds(i * 16, 16)] = cum - v + carry
        carry = carry + cum[15]

    ones16i = jnp.ones((16,), jnp.int32)

    def placeb(i, carry):
        enc = owned_v[pl.ds(i * 16, 16)]
        dl = jnp.bitwise_and(enc, 511)
        dlbuf[pl.ds(0, 16)] = dl
        rank = jnp.zeros((16,), jnp.int32)
        for sft in range(1, 16):
            msk = iota >= sft
            sh = plsc.load_gather(dlbuf, [iota - sft], mask=msk)
            eq = jnp.logical_and(sh == dl, msk)
            rank = rank + eq.astype(jnp.int32)
        pos = plsc.load_gather(off_v, [dl]) + rank
        plsc.store_scatter(sorted_v, [pos], enc)
        plsc.addupdate_scatter(off_v, [dl], ones16i)
        return carry

    lax.fori_loop(0, (cnt + 15) // 16, placeb, 0)

    misc_v[pl.ds(0, 16)] = jnp.full((16,), cnt, jnp.int32)
    pltpu.sync_copy(sorted_v.at[pl.ds(0, _CAP)], owned_hbm.at[wid, 0])
    pltpu.sync_copy(misc_v.at[pl.ds(0, 16)], cnt_hbm.at[wid, 0])
    pltpu.sync_copy(deg_v, deg_hbm.at[wid, 0])


def _make_reduce(p):
    @functools.partial(
        pl.kernel,
        out_type=[
            jax.ShapeDtypeStruct((_NPAD, 128), jnp.float32),
            jax.ShapeDtypeStruct((_NPAD, 128), jnp.float32),
        ],
        mesh=_sc_mesh,
        scratch_types=[
            pltpu.VMEM((_SPR, 128), jnp.float32),
            pltpu.VMEM((_SPR, 128), jnp.float32),
            pltpu.VMEM((2, _G, 128), jnp.float32),
            pltpu.VMEM((2, _G), jnp.int32),
            pltpu.VMEM((2, _G), jnp.int32),
            pltpu.VMEM((16,), jnp.int32),
            pltpu.SemaphoreType.DMA,
            pltpu.SemaphoreType.DMA,
        ],
        compiler_params=_sc_params,
    )
    def _reduce_p(xcat_hbm, owned_hbm, cnt_hbm, s2_hbm, mm_hbm,
                  sum_v, mm_v, gbuf, encb, srcb, cntv, gsem0, gsem1):
        wid = _wid()
        gsem = (gsem0, gsem1)
        obase = wid * _NPT

        pltpu.sync_copy(cnt_hbm.at[wid, 0], cntv)
        cnt = cntv[pl.ds(0, 16)][0]
        nch = (cnt + _G - 1) // _G

        ninf = jnp.full((16,), -jnp.inf, jnp.float32)
        pinf = jnp.full((16,), jnp.inf, jnp.float32)
        zero16 = jnp.zeros((16,), jnp.float32)

        def gather_cp(slot):
            return pltpu.make_async_copy(xcat_hbm.at[srcb.at[slot]],
                                         gbuf.at[slot], gsem[slot])

        def initb(i, carry):
            for k in range(4):
                sum_v[i, pl.ds(k * 16, 16)] = zero16
                sum_v[i, pl.ds(64 + k * 16, 16)] = zero16
                mm_v[i, pl.ds(k * 16, 16)] = ninf
                mm_v[i, pl.ds(64 + k * 16, 16)] = pinf
            return carry

        lax.fori_loop(0, _SPR, initb, 0)

        def prep(g, slot):
            pltpu.sync_copy(owned_hbm.at[wid, 0, pl.ds(g * _G, _G)],
                            encb.at[slot])
            for q in range(_G // 16):
                enc = encb[slot, pl.ds(q * 16, 16)]
                srcb[slot, pl.ds(q * 16, 16)] = jnp.right_shift(enc, 9)

        @pl.when(nch >= 1)
        def _():
            prep(0, 0)
            gather_cp(0).start()

        def outer(gg, carry):
            for b in range(2):
                g = gg * 2 + b
                slot, nxt = b, 1 - b

                @pl.when(g < nch)
                def _():
                    @pl.when(g + 1 < nch)
                    def _():
                        prep(g + 1, nxt)
                        gather_cp(nxt).start()

                    gather_cp(slot).wait()

                    def merge(cur, regs):
                        ss, qq, xx, nn = regs
                        for k in range(4):
                            cs = pl.ds(k * 16, 16)
                            c2 = pl.ds(64 + k * 16, 16)
                            sum_v[cur, cs] = sum_v[cur, cs] + ss[k]
                            sum_v[cur, c2] = sum_v[cur, c2] + qq[k]
                            mm_v[cur, cs] = jnp.maximum(mm_v[cur, cs], xx[k])
                            mm_v[cur, c2] = jnp.minimum(mm_v[cur, c2], nn[k])

                    neutral = ((zero16,) * 4, (zero16,) * 4,
                               (ninf,) * 4, (pinf,) * 4)

                    def rmw(q, carry):
                        cur, ss, qq, xx, nn = carry
                        ss, qq, xx, nn = (list(ss), list(qq),
                                          list(xx), list(nn))
                        enc = encb[slot, pl.ds(q * 16, 16)]
                        for j in range(16):
                            d = jnp.bitwise_and(enc[j], 511)
                            row = q * 16 + j
                            fl = d != cur

                            @pl.when(fl)
                            def _():
                                merge(cur, (ss, qq, xx, nn))

                            for k in range(4):
                                cg = pl.ds(64 * p + k * 16, 16)
                                m = gbuf[slot, row, cg]
                                m2 = m * m
                                ss[k] = jnp.where(fl, m, ss[k] + m)
                                qq[k] = jnp.where(fl, m2, qq[k] + m2)
                                xx[k] = jnp.where(
                                    fl, m, jnp.maximum(xx[k], m))
                                nn[k] = jnp.where(
                                    fl, m, jnp.minimum(nn[k], m))
                            cur = d
                        return (cur, tuple(ss), tuple(qq),
                                tuple(xx), tuple(nn))

                    fcur, fss, fqq, fxx, fnn = lax.fori_loop(
                        0, _G // 16, rmw,
                        (jnp.int32(_PAD_ENC),) + neutral)
                    merge(fcur, (fss, fqq, fxx, fnn))

            return carry

        lax.fori_loop(0, (nch + 1) // 2, outer, 0)

        pltpu.sync_copy(sum_v.at[pl.ds(0, _NPT)],
                        s2_hbm.at[pl.ds(obase, _NPT)])
        pltpu.sync_copy(mm_v.at[pl.ds(0, _NPT)],
                        mm_hbm.at[pl.ds(obase, _NPT)])

    return _reduce_p


_reduce_half = (_make_reduce(0), _make_reduce(1))


_BLK = 1024
_GRID = _NPAD // _BLK


def _dense_body(final, s2a_ref, s2b_ref, mma_ref, mmb_ref, deg_ref, h_ref,
                wa_ref, wb_ref, wc_ref, cb_ref, bn_ref, fcw_ref, fcb_ref,
                wih_ref, whh_ref, bih_ref, bhh_ref, lw_ref, lb_ref,
                hcat_ref, out_ref):
    s2a, s2b = s2a_ref[...], s2b_ref[...]
    mma, mmb = mma_ref[...], mmb_ref[...]
    ssum = jnp.concatenate([s2a[:, :64], s2b[:, :64]], axis=1)
    ssq = jnp.concatenate([s2a[:, 64:], s2b[:, 64:]], axis=1)
    mxr = jnp.concatenate([mma[:, :64], mmb[:, :64]], axis=1)
    mnr = jnp.concatenate([mma[:, 64:], mmb[:, 64:]], axis=1)
    deg = deg_ref[...]
    degc = jnp.maximum(deg, 1.0)
    inv = 1.0 / degc
    mean = ssum * inv
    var = jnp.maximum(ssq * inv - mean * mean, 0.0)
    std = jnp.sqrt(var + 1e-5)
    has = deg > 0.0
    mx = jnp.where(has, mxr, 0.0)
    mn = jnp.where(has, mnr, 0.0)
    agg = jnp.concatenate([mean, mn, mx, std], axis=1)
    logd = jnp.log(degc + 1.0)
    s1 = logd * (1.0 / _AVG_LOG)
    s2 = _AVG_LOG / logd
    x = (jnp.dot(agg, wa_ref[...], preferred_element_type=jnp.float32)
         + jnp.dot(agg * s1, wb_ref[...], preferred_element_type=jnp.float32)
         + jnp.dot(agg * s2, wc_ref[...], preferred_element_type=jnp.float32)
         + cb_ref[...])
    x = x * bn_ref[0:1, :] + bn_ref[1:2, :]
    x = jnp.dot(x, fcw_ref[...], preferred_element_type=jnp.float32) + fcb_ref[...]
    x = jnp.maximum(x, 0.0)
    h = h_ref[...]
    gi = jnp.dot(x, wih_ref[...], preferred_element_type=jnp.float32) + bih_ref[...]
    gh = jnp.dot(h, whh_ref[...], preferred_element_type=jnp.float32) + bhh_ref[...]
    i_r, i_z, i_n = gi[:, :HID], gi[:, HID:2 * HID], gi[:, 2 * HID:]
    h_r, h_z, h_n = gh[:, :HID], gh[:, HID:2 * HID], gh[:, 2 * HID:]
    r = jax.nn.sigmoid(i_r + h_r)
    z = jax.nn.sigmoid(i_z + h_z)
    ng = jnp.tanh(i_n + r * h_n)
    hnew = (1.0 - z) * ng + z * h
    hcat_ref[...] = hnew
    if final:
        out_ref[...] = (jnp.dot(hnew, lw_ref[...],
                                preferred_element_type=jnp.float32) + lb_ref[...])


def _node_spec(cols):
    return pl.BlockSpec((_BLK, cols), lambda i: (i, 0))


def _rep_spec(shape):
    nd = len(shape)
    return pl.BlockSpec(shape, lambda i: (0,) * nd)


@functools.partial(jax.jit, static_argnames=("final",))
def _dense_layer(s2a, s2b, mma, mmb, deg, h, wa, wb, wc, cb, bn, fcw, fcb,
                 wih, whh, bih, bhh, lw, lb, final=False):
    out_shapes = [jax.ShapeDtypeStruct((_NPAD, HID), jnp.float32),
                  jax.ShapeDtypeStruct((_NPAD, HID), jnp.float32)]
    in_specs = [_node_spec(HID)] * 4 + [_node_spec(1), _node_spec(HID)]
    in_specs += [_rep_spec(w.shape) for w in
                 (wa, wb, wc, cb, bn, fcw, fcb, wih, whh, bih, bhh, lw, lb)]
    return pl.pallas_call(
        functools.partial(_dense_body, final),
        grid=(_GRID,),
        in_specs=in_specs,
        out_specs=[_node_spec(HID), _node_spec(HID)],
        out_shape=out_shapes,
    )(s2a, s2b, mma, mmb, deg, h, wa, wb, wc, cb, bn, fcw, fcb,
      wih, whh, bih, bhh, lw, lb)


def kernel(x, edge_index, batch, params):
    src = edge_index[0]
    dst = edge_index[1]
    owned, cnt2, degw = _route(src, dst)
    deg = degw[:, 0, :_NPT].reshape(-1)[:, None]

    xcat = jnp.pad(x, ((0, _NPAD - N_NODES), (0, 0)))
    h = jnp.zeros((_NPAD, HID), jnp.float32)
    bih = params['gru_b_ih'][None, :]
    bhh = params['gru_b_hh'][None, :]
    wih = params['gru_w_ih'].T
    whh = params['gru_w_hh'].T
    lw = params['last_w'].T
    lb = params['last_b'][None, :]
    out = None
    for i in range(NUM_LAYERS):
        (s2a, mma), (s2b, mmb) = [
            _reduce_half[p](xcat, owned, cnt2) for p in range(2)]
        w = params['conv%d_w' % i]
        wa = w[:, :512].T
        wb = w[:, 512:1024].T
        wc = w[:, 1024:].T
        cb = params['conv%d_b' % i][None, :]
        g = params['bn%d_g' % i] / math.sqrt(1.0 + 1e-5)
        bn = jnp.stack([g, params['bn%d_b' % i]], axis=0)
        fcw = params['fc%d_w' % i].T
        fcb = params['fc%d_b' % i][None, :]
        final = i == NUM_LAYERS - 1
        hcat, out = _dense_layer(
            s2a, s2b, mma, mmb, deg, h, wa, wb, wc, cb, bn, fcw, fcb,
            wih, whh, bih, bhh, lw, lb, final=final)
        h = hcat
        if not final:
            xcat = h
    return out[:N_NODES]
